# Initial kernel scaffold; baseline (speedup 1.0000x reference)
import jax
import jax.numpy as jnp
from jax.experimental import pallas as pl


def kernel(x, edge_index, batch, W1m, b1m, W1u, b1u, W2m, b2m, W2u, b2u, Wc, bc):
    raise NotImplementedError("write your pallas kernel here")


# speedup vs baseline: 3.5388x; 3.5388x over previous
import functools

import jax
import jax.numpy as jnp
from jax import lax
from jax.experimental import pallas as pl
from jax.experimental.pallas import tpu as pltpu
from jax.experimental.pallas import tpu_sc as plsc

F = 128
N_NODES = 10000
N_PAD = 10240
N_EDGES = 320000
G = 64
NCLS = 10

NC = 2
NS = 16
L = 16

NPC = N_PAD // NC
EPT = N_EDGES // NS
EK = 80
ESTEPS = EPT // EK
EG = 25
NGRP = ESTEPS // EG
NPT = NPC // NS
ZR = 64

PACT = 25
PNP = N_NODES // PACT
PK = 80
PSTEPS = PNP // PK

_f32 = jnp.float32


def _fill(ref, rows, cols, value):
    def body(i, _):
        for j in range(cols // L):
            ref[i, pl.ds(j * L, L)] = jnp.full((L,), value, _f32)
        return 0
    lax.fori_loop(0, rows, body, 0)


def _sc_aggregate(feat, row4d, col4d, with_deg):
    out_types = [jax.ShapeDtypeStruct((N_PAD, F), _f32)]
    scratch = [
        pltpu.VMEM((EG, EK), jnp.int32),
        pltpu.VMEM((EG, EK), jnp.int32),
        pltpu.VMEM((EK, F), _f32),
        pltpu.VMEM((ZR, F), _f32),
        pltpu.VMEM_SHARED((NPC + 8, F), _f32),
        pltpu.SemaphoreType.DMA,
    ]
    if with_deg:
        out_types.append(jax.ShapeDtypeStruct((N_PAD, L), _f32))
        scratch += [
            pltpu.VMEM((EK, L), _f32),
            pltpu.VMEM((ZR, L), _f32),
            pltpu.VMEM_SHARED((NPC + 8, L), _f32),
        ]

    mesh = plsc.VectorSubcoreMesh(core_axis_name="c", subcore_axis_name="s")

    def body(feat_hbm, row_hbm, col_hbm, *rest):
        if with_deg:
            (a_out, d_out, ridx, cidx, rows, zbuf, a_sh, sem,
             ones, zbufd, d_sh) = rest
        else:
            a_out, ridx, cidx, rows, zbuf, a_sh, sem = rest
        c = lax.axis_index("c")
        s = lax.axis_index("s")

        _fill(zbuf, ZR, F, 0.0)
        for z in range(NPT // ZR):
            pltpu.sync_copy(zbuf, a_sh.at[pl.ds(s * NPT + z * ZR, ZR)])
        if with_deg:
            _fill(ones, EK, L, 1.0)
            _fill(zbufd, ZR, L, 0.0)
            for z in range(NPT // ZR):
                pltpu.sync_copy(zbufd, d_sh.at[pl.ds(s * NPT + z * ZR, ZR)])

        plsc.subcore_barrier()

        base = c * NPC

        def group(g, _):
            pltpu.sync_copy(row_hbm.at[s, g], ridx)
            pltpu.sync_copy(col_hbm.at[s, g], cidx)

            def loc(i, _):
                for j in range(EK // L):
                    v = ridx[i, pl.ds(j * L, L)] - base
                    ok = (v >= 0) & (v < NPC)
                    ridx[i, pl.ds(j * L, L)] = jnp.where(ok, v, NPC)
                return 0
            lax.fori_loop(0, EG, loc, 0)

            def step(t, _):
                pltpu.async_copy(feat_hbm.at[cidx.at[t]], rows, sem).wait()
                pltpu.sync_copy(rows, a_sh.at[ridx.at[t]], add=True)
                if with_deg:
                    pltpu.sync_copy(ones, d_sh.at[ridx.at[t]], add=True)
                return 0
            lax.fori_loop(0, EG, step, 0)
            return 0
        lax.fori_loop(0, NGRP, group, 0)

        plsc.subcore_barrier()

        r0 = s * NPT
        pltpu.sync_copy(a_sh.at[pl.ds(r0, NPT)],
                        a_out.at[pl.ds(c * NPC + r0, NPT)])
        if with_deg:
            pltpu.sync_copy(d_sh.at[pl.ds(r0, NPT)],
                            d_out.at[pl.ds(c * NPC + r0, NPT)])

    fn = functools.partial(
        pl.kernel, mesh=mesh,
        out_type=tuple(out_types) if len(out_types) > 1 else out_types[0],
        scratch_types=scratch,
    )(body)
    return fn(feat, row4d, col4d)


def _sc_pool(h, batch3d):
    mesh = plsc.VectorSubcoreMesh(core_axis_name="c", subcore_axis_name="s")

    def body(h_hbm, b_hbm, p_out, c_out, bidx, rows, ones, zp, zc, p_sh, c_sh,
             sem):
        c = lax.axis_index("c")
        s = lax.axis_index("s")
        wid = s * NC + c

        @pl.when(s == 0)
        def _zero():
            _fill(zp, G, F, 0.0)
            _fill(zc, G, L, 0.0)
            pltpu.sync_copy(zp, p_sh)
            pltpu.sync_copy(zc, c_sh)

        plsc.subcore_barrier()

        @pl.when(wid < PACT)
        def _accum():
            _fill(ones, PK, L, 1.0)
            pltpu.sync_copy(b_hbm.at[wid], bidx)

            def step(t, _):
                pltpu.async_copy(
                    h_hbm.at[pl.ds(wid * PNP + t * PK, PK)], rows, sem).wait()
                pltpu.sync_copy(rows, p_sh.at[bidx.at[t]], add=True)
                pltpu.sync_copy(ones, c_sh.at[bidx.at[t]], add=True)
                return 0
            lax.fori_loop(0, PSTEPS, step, 0)

        plsc.subcore_barrier()

        @pl.when(s == 0)
        def _write():
            pltpu.sync_copy(p_sh, p_out.at[c])
            pltpu.sync_copy(c_sh, c_out.at[c])

    fn = functools.partial(
        pl.kernel, mesh=mesh,
        out_type=(jax.ShapeDtypeStruct((NC, G, F), _f32),
                  jax.ShapeDtypeStruct((NC, G, L), _f32)),
        scratch_types=[
            pltpu.VMEM((PSTEPS, PK), jnp.int32),
            pltpu.VMEM((PK, F), _f32),
            pltpu.VMEM((PK, L), _f32),
            pltpu.VMEM((G, F), _f32),
            pltpu.VMEM((G, L), _f32),
            pltpu.VMEM_SHARED((G, F), _f32),
            pltpu.VMEM_SHARED((G, L), _f32),
            pltpu.SemaphoreType.DMA,
        ],
    )(body)
    return fn(h, batch3d)


BN = 512
NB = N_PAD // BN


def _tc_layer(a, deg, xin, Wm, bm, Wu_top, Wu_bot, bu):
    def body(a_ref, d_ref, x_ref, wm_ref, bm_ref, ut_ref, ub_ref, bu_ref,
             o_ref):
        aggr = jnp.dot(a_ref[:], wm_ref[:], preferred_element_type=_f32)
        aggr = aggr + d_ref[:, 0:1] * bm_ref[:]
        h = (jnp.dot(x_ref[:], ut_ref[:], preferred_element_type=_f32)
             + jnp.dot(aggr, ub_ref[:], preferred_element_type=_f32)
             + bu_ref[:])
        o_ref[:] = jnp.maximum(h, 0.0)

    return pl.pallas_call(
        body,
        grid=(NB,),
        in_specs=[
            pl.BlockSpec((BN, F), lambda i: (i, 0)),
            pl.BlockSpec((BN, L), lambda i: (i, 0)),
            pl.BlockSpec((BN, F), lambda i: (i, 0)),
            pl.BlockSpec((F, F), lambda i: (0, 0)),
            pl.BlockSpec((1, F), lambda i: (0, 0)),
            pl.BlockSpec((F, F), lambda i: (0, 0)),
            pl.BlockSpec((F, F), lambda i: (0, 0)),
            pl.BlockSpec((1, F), lambda i: (0, 0)),
        ],
        out_specs=pl.BlockSpec((BN, F), lambda i: (i, 0)),
        out_shape=jax.ShapeDtypeStruct((N_PAD, F), _f32),
    )(a, deg, xin, Wm, bm, Wu_top, Wu_bot, bu)


def _tc_head(p_part, c_part, Wc_pad, bc_pad):
    def body(p_ref, c_ref, wc_ref, bc_ref, o_ref):
        p = p_ref[0] + p_ref[1]
        cnt = c_ref[0, :, 0:1] + c_ref[1, :, 0:1]
        pooled = p / cnt
        o_ref[:] = (jnp.dot(pooled, wc_ref[:], preferred_element_type=_f32)
                    + bc_ref[:])

    return pl.pallas_call(
        body,
        out_shape=jax.ShapeDtypeStruct((G, F), _f32),
    )(p_part, c_part, Wc_pad, bc_pad)


def kernel(x, edge_index, batch, W1m, b1m, W1u, b1u, W2m, b2m, W2u, b2u,
           Wc, bc):
    row = edge_index[0].astype(jnp.int32).reshape(NS, NGRP, EG, EK)
    col = edge_index[1].astype(jnp.int32).reshape(NS, NGRP, EG, EK)
    batch3d = batch.astype(jnp.int32).reshape(PACT, PSTEPS, PK)

    b1m_2d = b1m.reshape(1, F)
    b1u_2d = b1u.reshape(1, F)
    b2m_2d = b2m.reshape(1, F)
    b2u_2d = b2u.reshape(1, F)
    Wc_pad = jnp.zeros((F, F), _f32).at[:, :NCLS].set(Wc)
    bc_pad = jnp.zeros((1, F), _f32).at[0, :NCLS].set(bc)

    xp = jnp.zeros((N_PAD, F), _f32).at[:N_NODES].set(x)

    a1, deg = _sc_aggregate(xp, row, col, with_deg=True)
    h1 = _tc_layer(a1, deg, xp, W1m, b1m_2d, W1u[:F], W1u[F:], b1u_2d)
    a2 = _sc_aggregate(h1, row, col, with_deg=False)
    h2 = _tc_layer(a2, deg, h1, W2m, b2m_2d, W2u[:F], W2u[F:], b2u_2d)
    p, cnt = _sc_pool(h2, batch3d)
    out = _tc_head(p, cnt, Wc_pad, bc_pad)
    return out[:, :NCLS]


# speedup vs baseline: 5.5053x; 1.5557x over previous
import functools

import jax
import jax.numpy as jnp
from jax import lax
from jax.experimental import pallas as pl
from jax.experimental.pallas import tpu as pltpu
from jax.experimental.pallas import tpu_sc as plsc

F = 128
N_NODES = 10000
N_PAD = 10240
N_EDGES = 320000
G = 64
NCLS = 10

NC = 2
NS = 16
NW = NC * NS
L = 16

EPT = N_EDGES // NW
EK = 80
ESTEPS = EPT // EK
EG = 5
NGRP = ESTEPS // EG
NPT = N_PAD // NS
ZR = 16

PACT = 25
PNP = N_NODES // PACT
PK = 80
PSTEPS = PNP // PK

_f32 = jnp.float32


def _fill(ref, rows, cols, value):
    def body(i, _):
        for j in range(cols // L):
            ref[i, pl.ds(j * L, L)] = jnp.full((L,), value, _f32)
        return 0
    lax.fori_loop(0, rows, body, 0)


def _sc_degree(row4d):
    mesh = plsc.VectorSubcoreMesh(core_axis_name="c", subcore_axis_name="s")

    def body(row_hbm, d_out, idx, ones, d_sh):
        c = lax.axis_index("c")
        s = lax.axis_index("s")
        wid = c * NS + s

        _fill(ones, EK, L, 0.0)
        for z in range(NPT // EK):
            pltpu.sync_copy(ones, d_sh.at[pl.ds(s * NPT + z * EK, EK)])
        _fill(ones, EK, L, 1.0)

        plsc.subcore_barrier()

        def group(g, _):
            pltpu.sync_copy(row_hbm.at[wid, g], idx)

            def step(t, _):
                pltpu.sync_copy(ones, d_sh.at[idx.at[t]], add=True)
                return 0
            lax.fori_loop(0, EG, step, 0)
            return 0
        lax.fori_loop(0, NGRP, group, 0)

        plsc.subcore_barrier()

        r0 = s * NPT
        pltpu.sync_copy(d_sh.at[pl.ds(r0, NPT)], d_out.at[c, pl.ds(r0, NPT)])

    fn = functools.partial(
        pl.kernel, mesh=mesh,
        out_type=jax.ShapeDtypeStruct((NC, N_PAD, L), _f32),
        scratch_types=[
            pltpu.VMEM((EG, EK), jnp.int32),
            pltpu.VMEM((EK, L), _f32),
            pltpu.VMEM_SHARED((N_PAD, L), _f32),
        ],
    )(body)
    return fn(row4d)


def _sc_aggregate(feat, row4d, col4d):
    mesh = plsc.VectorSubcoreMesh(core_axis_name="c", subcore_axis_name="s")

    def body(feat_hbm, row_hbm, col_hbm, a_out, ridx, cidx, rows, zbuf, a_sh,
             sem):
        c = lax.axis_index("c")
        s = lax.axis_index("s")
        wid = c * NS + s

        _fill(zbuf, ZR, F, 0.0)
        for z in range(NPT // ZR):
            pltpu.sync_copy(zbuf, a_sh.at[pl.ds(s * NPT + z * ZR, ZR)])

        plsc.subcore_barrier()

        def group(g, _):
            pltpu.sync_copy(row_hbm.at[wid, g], ridx)
            pltpu.sync_copy(col_hbm.at[wid, g], cidx)

            def step(t, _):
                pltpu.async_copy(feat_hbm.at[cidx.at[t]], rows, sem).wait()
                pltpu.sync_copy(rows, a_sh.at[ridx.at[t]], add=True)
                return 0
            lax.fori_loop(0, EG, step, 0)
            return 0
        lax.fori_loop(0, NGRP, group, 0)

        plsc.subcore_barrier()

        r0 = s * NPT
        pltpu.sync_copy(a_sh.at[pl.ds(r0, NPT)], a_out.at[c, pl.ds(r0, NPT)])

    fn = functools.partial(
        pl.kernel, mesh=mesh,
        out_type=jax.ShapeDtypeStruct((NC, N_PAD, F), _f32),
        scratch_types=[
            pltpu.VMEM((EG, EK), jnp.int32),
            pltpu.VMEM((EG, EK), jnp.int32),
            pltpu.VMEM((EK, F), _f32),
            pltpu.VMEM((ZR, F), _f32),
            pltpu.VMEM_SHARED((N_PAD, F), _f32),
            pltpu.SemaphoreType.DMA,
        ],
    )(body)
    return fn(feat, row4d, col4d)


def _sc_pool(h, batch3d):
    mesh = plsc.VectorSubcoreMesh(core_axis_name="c", subcore_axis_name="s")

    def body(h_hbm, b_hbm, p_out, c_out, bidx, rows, ones, zp, zc, p_sh, c_sh,
             sem):
        c = lax.axis_index("c")
        s = lax.axis_index("s")
        wid = s * NC + c

        @pl.when(s == 0)
        def _zero():
            _fill(zp, G, F, 0.0)
            _fill(zc, G, L, 0.0)
            pltpu.sync_copy(zp, p_sh)
            pltpu.sync_copy(zc, c_sh)

        plsc.subcore_barrier()

        @pl.when(wid < PACT)
        def _accum():
            _fill(ones, PK, L, 1.0)
            pltpu.sync_copy(b_hbm.at[wid], bidx)

            def step(t, _):
                pltpu.async_copy(
                    h_hbm.at[pl.ds(wid * PNP + t * PK, PK)], rows, sem).wait()
                pltpu.sync_copy(rows, p_sh.at[bidx.at[t]], add=True)
                pltpu.sync_copy(ones, c_sh.at[bidx.at[t]], add=True)
                return 0
            lax.fori_loop(0, PSTEPS, step, 0)

        plsc.subcore_barrier()

        @pl.when(s == 0)
        def _write():
            pltpu.sync_copy(p_sh, p_out.at[c])
            pltpu.sync_copy(c_sh, c_out.at[c])

    fn = functools.partial(
        pl.kernel, mesh=mesh,
        out_type=(jax.ShapeDtypeStruct((NC, G, F), _f32),
                  jax.ShapeDtypeStruct((NC, G, L), _f32)),
        scratch_types=[
            pltpu.VMEM((PSTEPS, PK), jnp.int32),
            pltpu.VMEM((PK, F), _f32),
            pltpu.VMEM((PK, L), _f32),
            pltpu.VMEM((G, F), _f32),
            pltpu.VMEM((G, L), _f32),
            pltpu.VMEM_SHARED((G, F), _f32),
            pltpu.VMEM_SHARED((G, L), _f32),
            pltpu.SemaphoreType.DMA,
        ],
    )(body)
    return fn(h, batch3d)


BN = 512
NB = N_PAD // BN


def _tc_layer(a_part, deg_part, xin, Wm, bm, Wu_top, Wu_bot, bu):
    def body(a_ref, d_ref, x_ref, wm_ref, bm_ref, ut_ref, ub_ref, bu_ref,
             o_ref):
        a = a_ref[0] + a_ref[1]
        deg = d_ref[0, :, 0:1] + d_ref[1, :, 0:1]
        aggr = jnp.dot(a, wm_ref[:], preferred_element_type=_f32)
        aggr = aggr + deg * bm_ref[:]
        h = (jnp.dot(x_ref[:], ut_ref[:], preferred_element_type=_f32)
             + jnp.dot(aggr, ub_ref[:], preferred_element_type=_f32)
             + bu_ref[:])
        o_ref[:] = jnp.maximum(h, 0.0)

    return pl.pallas_call(
        body,
        grid=(NB,),
        in_specs=[
            pl.BlockSpec((NC, BN, F), lambda i: (0, i, 0)),
            pl.BlockSpec((NC, BN, L), lambda i: (0, i, 0)),
            pl.BlockSpec((BN, F), lambda i: (i, 0)),
            pl.BlockSpec((F, F), lambda i: (0, 0)),
            pl.BlockSpec((1, F), lambda i: (0, 0)),
            pl.BlockSpec((F, F), lambda i: (0, 0)),
            pl.BlockSpec((F, F), lambda i: (0, 0)),
            pl.BlockSpec((1, F), lambda i: (0, 0)),
        ],
        out_specs=pl.BlockSpec((BN, F), lambda i: (i, 0)),
        out_shape=jax.ShapeDtypeStruct((N_PAD, F), _f32),
    )(a_part, deg_part, xin, Wm, bm, Wu_top, Wu_bot, bu)


def _tc_head(p_part, c_part, Wc_pad, bc_pad):
    def body(p_ref, c_ref, wc_ref, bc_ref, o_ref):
        p = p_ref[0] + p_ref[1]
        cnt = c_ref[0, :, 0:1] + c_ref[1, :, 0:1]
        pooled = p / cnt
        o_ref[:] = (jnp.dot(pooled, wc_ref[:], preferred_element_type=_f32)
                    + bc_ref[:])

    return pl.pallas_call(
        body,
        out_shape=jax.ShapeDtypeStruct((G, F), _f32),
    )(p_part, c_part, Wc_pad, bc_pad)


def kernel(x, edge_index, batch, W1m, b1m, W1u, b1u, W2m, b2m, W2u, b2u,
           Wc, bc):
    row = edge_index[0].astype(jnp.int32).reshape(NW, NGRP, EG, EK)
    col = edge_index[1].astype(jnp.int32).reshape(NW, NGRP, EG, EK)
    batch3d = batch.astype(jnp.int32).reshape(PACT, PSTEPS, PK)

    b1m_2d = b1m.reshape(1, F)
    b1u_2d = b1u.reshape(1, F)
    b2m_2d = b2m.reshape(1, F)
    b2u_2d = b2u.reshape(1, F)
    Wc_pad = jnp.zeros((F, F), _f32).at[:, :NCLS].set(Wc)
    bc_pad = jnp.zeros((1, F), _f32).at[0, :NCLS].set(bc)

    xp = jnp.zeros((N_PAD, F), _f32).at[:N_NODES].set(x)

    deg = _sc_degree(row)
    a1 = _sc_aggregate(xp, row, col)
    h1 = _tc_layer(a1, deg, xp, W1m, b1m_2d, W1u[:F], W1u[F:], b1u_2d)
    a2 = _sc_aggregate(h1, row, col)
    h2 = _tc_layer(a2, deg, h1, W2m, b2m_2d, W2u[:F], W2u[F:], b2u_2d)
    p, cnt = _sc_pool(h2, batch3d)
    out = _tc_head(p, cnt, Wc_pad, bc_pad)
    return out[:, :NCLS]


# speedup vs baseline: 6.6090x; 1.2005x over previous
import functools

import jax
import jax.numpy as jnp
from jax import lax
from jax.experimental import pallas as pl
from jax.experimental.pallas import tpu as pltpu
from jax.experimental.pallas import tpu_sc as plsc

F = 128
N_NODES = 10000
N_PAD = 10240
N_EDGES = 320000
G = 64
NCLS = 10

NC = 2
NS = 16
NW = NC * NS
L = 16

EPT = N_EDGES // NW
EK = 128
EG = 6
NGRP = 13
TK = EPT - NGRP * EG * EK
NPT = N_PAD // NS

PACT = 25
PNP = N_NODES // PACT
PK = 80
PSTEPS = PNP // PK

_f32 = jnp.float32


def _fill(ref, rows, cols, value):
    def body(i, _):
        for j in range(cols // L):
            ref[i, pl.ds(j * L, L)] = jnp.full((L,), value, _f32)
        return 0
    lax.fori_loop(0, rows, body, 0)


def _sc_degree(rm, rt):
    mesh = plsc.VectorSubcoreMesh(core_axis_name="c", subcore_axis_name="s")

    def body(rm_hbm, rt_hbm, d_out, idx, tidx, ones, d_sh):
        c = lax.axis_index("c")
        s = lax.axis_index("s")
        wid = c * NS + s

        _fill(ones, EK, L, 0.0)
        for z in range(NPT // EK):
            pltpu.sync_copy(ones, d_sh.at[pl.ds(s * NPT + z * EK, EK)])
        _fill(ones, EK, L, 1.0)

        plsc.subcore_barrier()

        def group(g, _):
            pltpu.sync_copy(rm_hbm.at[wid, g], idx)

            def step(t, _):
                pltpu.sync_copy(ones, d_sh.at[idx.at[t]], add=True)
                return 0
            lax.fori_loop(0, EG, step, 0)
            return 0
        lax.fori_loop(0, NGRP, group, 0)

        pltpu.sync_copy(rt_hbm.at[wid], tidx)
        pltpu.sync_copy(ones.at[pl.ds(0, TK)], d_sh.at[tidx], add=True)

        plsc.subcore_barrier()

        r0 = s * NPT
        pltpu.sync_copy(d_sh.at[pl.ds(r0, NPT)], d_out.at[c, pl.ds(r0, NPT)])

    fn = functools.partial(
        pl.kernel, mesh=mesh,
        out_type=jax.ShapeDtypeStruct((NC, N_PAD, L), _f32),
        scratch_types=[
            pltpu.VMEM((EG, EK), jnp.int32),
            pltpu.VMEM((TK,), jnp.int32),
            pltpu.VMEM((EK, L), _f32),
            pltpu.VMEM_SHARED((N_PAD, L), _f32),
        ],
    )(body)
    return fn(rm, rt)


def _sc_aggregate(feat, rm, cm, rt, ct):
    mesh = plsc.VectorSubcoreMesh(core_axis_name="c", subcore_axis_name="s")

    def body(feat_hbm, rm_hbm, cm_hbm, rt_hbm, ct_hbm, a_out,
             ridx, cidx, tridx, tcidx, rows, a_sh, sem):
        c = lax.axis_index("c")
        s = lax.axis_index("s")
        wid = c * NS + s

        _fill(rows, EK, F, 0.0)
        for z in range(NPT // EK):
            pltpu.sync_copy(rows, a_sh.at[pl.ds(s * NPT + z * EK, EK)])

        plsc.subcore_barrier()

        def group(g, _):
            pltpu.sync_copy(rm_hbm.at[wid, g], ridx)
            pltpu.sync_copy(cm_hbm.at[wid, g], cidx)

            def step(t, _):
                pltpu.async_copy(feat_hbm.at[cidx.at[t]], rows, sem).wait()
                pltpu.sync_copy(rows, a_sh.at[ridx.at[t]], add=True)
                return 0
            lax.fori_loop(0, EG, step, 0)
            return 0
        lax.fori_loop(0, NGRP, group, 0)

        pltpu.sync_copy(rt_hbm.at[wid], tridx)
        pltpu.sync_copy(ct_hbm.at[wid], tcidx)
        pltpu.async_copy(feat_hbm.at[tcidx], rows.at[pl.ds(0, TK)],
                         sem).wait()
        pltpu.sync_copy(rows.at[pl.ds(0, TK)], a_sh.at[tridx], add=True)

        plsc.subcore_barrier()

        r0 = s * NPT
        pltpu.sync_copy(a_sh.at[pl.ds(r0, NPT)], a_out.at[c, pl.ds(r0, NPT)])

    fn = functools.partial(
        pl.kernel, mesh=mesh,
        out_type=jax.ShapeDtypeStruct((NC, N_PAD, F), _f32),
        scratch_types=[
            pltpu.VMEM((EG, EK), jnp.int32),
            pltpu.VMEM((EG, EK), jnp.int32),
            pltpu.VMEM((TK,), jnp.int32),
            pltpu.VMEM((TK,), jnp.int32),
            pltpu.VMEM((EK, F), _f32),
            pltpu.VMEM_SHARED((N_PAD, F), _f32),
            pltpu.SemaphoreType.DMA,
        ],
    )(body)
    return fn(feat, rm, cm, rt, ct)


def _sc_pool(h, batch3d):
    mesh = plsc.VectorSubcoreMesh(core_axis_name="c", subcore_axis_name="s")

    def body(h_hbm, b_hbm, p_out, c_out, bidx, rows, ones, zp, zc, p_sh, c_sh,
             sem):
        c = lax.axis_index("c")
        s = lax.axis_index("s")
        wid = s * NC + c

        @pl.when(s == 0)
        def _zero():
            _fill(zp, G, F, 0.0)
            _fill(zc, G, L, 0.0)
            pltpu.sync_copy(zp, p_sh)
            pltpu.sync_copy(zc, c_sh)

        plsc.subcore_barrier()

        @pl.when(wid < PACT)
        def _accum():
            _fill(ones, PK, L, 1.0)
            pltpu.sync_copy(b_hbm.at[wid], bidx)

            def step(t, _):
                pltpu.async_copy(
                    h_hbm.at[pl.ds(wid * PNP + t * PK, PK)], rows, sem).wait()
                pltpu.sync_copy(rows, p_sh.at[bidx.at[t]], add=True)
                pltpu.sync_copy(ones, c_sh.at[bidx.at[t]], add=True)
                return 0
            lax.fori_loop(0, PSTEPS, step, 0)

        plsc.subcore_barrier()

        @pl.when(s == 0)
        def _write():
            pltpu.sync_copy(p_sh, p_out.at[c])
            pltpu.sync_copy(c_sh, c_out.at[c])

    fn = functools.partial(
        pl.kernel, mesh=mesh,
        out_type=(jax.ShapeDtypeStruct((NC, G, F), _f32),
                  jax.ShapeDtypeStruct((NC, G, L), _f32)),
        scratch_types=[
            pltpu.VMEM((PSTEPS, PK), jnp.int32),
            pltpu.VMEM((PK, F), _f32),
            pltpu.VMEM((PK, L), _f32),
            pltpu.VMEM((G, F), _f32),
            pltpu.VMEM((G, L), _f32),
            pltpu.VMEM_SHARED((G, F), _f32),
            pltpu.VMEM_SHARED((G, L), _f32),
            pltpu.SemaphoreType.DMA,
        ],
    )(body)
    return fn(h, batch3d)


BN = 512
NB = N_PAD // BN


def _tc_layer(a_part, deg_part, xin, Wm, bm, Wu_top, Wu_bot, bu):
    def body(a_ref, d_ref, x_ref, wm_ref, bm_ref, ut_ref, ub_ref, bu_ref,
             o_ref):
        a = a_ref[0] + a_ref[1]
        deg = d_ref[0, :, 0:1] + d_ref[1, :, 0:1]
        aggr = jnp.dot(a, wm_ref[:], preferred_element_type=_f32)
        aggr = aggr + deg * bm_ref[:]
        h = (jnp.dot(x_ref[:], ut_ref[:], preferred_element_type=_f32)
             + jnp.dot(aggr, ub_ref[:], preferred_element_type=_f32)
             + bu_ref[:])
        o_ref[:] = jnp.maximum(h, 0.0)

    return pl.pallas_call(
        body,
        grid=(NB,),
        in_specs=[
            pl.BlockSpec((NC, BN, F), lambda i: (0, i, 0)),
            pl.BlockSpec((NC, BN, L), lambda i: (0, i, 0)),
            pl.BlockSpec((BN, F), lambda i: (i, 0)),
            pl.BlockSpec((F, F), lambda i: (0, 0)),
            pl.BlockSpec((1, F), lambda i: (0, 0)),
            pl.BlockSpec((F, F), lambda i: (0, 0)),
            pl.BlockSpec((F, F), lambda i: (0, 0)),
            pl.BlockSpec((1, F), lambda i: (0, 0)),
        ],
        out_specs=pl.BlockSpec((BN, F), lambda i: (i, 0)),
        out_shape=jax.ShapeDtypeStruct((N_PAD, F), _f32),
    )(a_part, deg_part, xin, Wm, bm, Wu_top, Wu_bot, bu)


def _tc_head(p_part, c_part, Wc_pad, bc_pad):
    def body(p_ref, c_ref, wc_ref, bc_ref, o_ref):
        p = p_ref[0] + p_ref[1]
        cnt = c_ref[0, :, 0:1] + c_ref[1, :, 0:1]
        pooled = p / cnt
        o_ref[:] = (jnp.dot(pooled, wc_ref[:], preferred_element_type=_f32)
                    + bc_ref[:])

    return pl.pallas_call(
        body,
        out_shape=jax.ShapeDtypeStruct((G, F), _f32),
    )(p_part, c_part, Wc_pad, bc_pad)


def kernel(x, edge_index, batch, W1m, b1m, W1u, b1u, W2m, b2m, W2u, b2u,
           Wc, bc):
    row2 = edge_index[0].astype(jnp.int32).reshape(NW, EPT)
    col2 = edge_index[1].astype(jnp.int32).reshape(NW, EPT)
    nmain = NGRP * EG * EK
    rm = row2[:, :nmain].reshape(NW, NGRP, EG, EK)
    cm = col2[:, :nmain].reshape(NW, NGRP, EG, EK)
    rt = row2[:, nmain:]
    ct = col2[:, nmain:]
    batch3d = batch.astype(jnp.int32).reshape(PACT, PSTEPS, PK)

    b1m_2d = b1m.reshape(1, F)
    b1u_2d = b1u.reshape(1, F)
    b2m_2d = b2m.reshape(1, F)
    b2u_2d = b2u.reshape(1, F)
    Wc_pad = jnp.zeros((F, F), _f32).at[:, :NCLS].set(Wc)
    bc_pad = jnp.zeros((1, F), _f32).at[0, :NCLS].set(bc)

    xp = jnp.zeros((N_PAD, F), _f32).at[:N_NODES].set(x)

    deg = _sc_degree(rm, rt)
    a1 = _sc_aggregate(xp, rm, cm, rt, ct)
    h1 = _tc_layer(a1, deg, xp, W1m, b1m_2d, W1u[:F], W1u[F:], b1u_2d)
    a2 = _sc_aggregate(h1, rm, cm, rt, ct)
    h2 = _tc_layer(a2, deg, h1, W2m, b2m_2d, W2u[:F], W2u[F:], b2u_2d)
    p, cnt = _sc_pool(h2, batch3d)
    out = _tc_head(p, cnt, Wc_pad, bc_pad)
    return out[:, :NCLS]


# speedup vs baseline: 6.8250x; 1.0327x over previous
import functools

import jax
import jax.numpy as jnp
from jax import lax
from jax.experimental import pallas as pl
from jax.experimental.pallas import tpu as pltpu
from jax.experimental.pallas import tpu_sc as plsc

F = 128
N_NODES = 10000
N_PAD = 10240
N_EDGES = 320000
G = 64
NCLS = 10

NC = 2
NS = 16
NW = NC * NS
L = 16

EPT = N_EDGES // NW
EK = 128
EG = 13
NGRP = 6
TK = EPT - NGRP * EG * EK
NPT = N_PAD // NS

PACT = 25
PNP = N_NODES // PACT
PK = 80
PSTEPS = PNP // PK

_f32 = jnp.float32


def _fill(ref, rows, cols, value):
    def body(i, _):
        for j in range(cols // L):
            ref[i, pl.ds(j * L, L)] = jnp.full((L,), value, _f32)
        return 0
    lax.fori_loop(0, rows, body, 0)


def _sc_degree(rm, rt):
    mesh = plsc.VectorSubcoreMesh(core_axis_name="c", subcore_axis_name="s")

    def body(rm_hbm, rt_hbm, d_out, idx, tidx, ones, d_sh):
        c = lax.axis_index("c")
        s = lax.axis_index("s")
        wid = c * NS + s

        _fill(ones, EK, L, 0.0)
        for z in range(NPT // EK):
            pltpu.sync_copy(ones, d_sh.at[pl.ds(s * NPT + z * EK, EK)])
        _fill(ones, EK, L, 1.0)

        plsc.subcore_barrier()

        def group(g, _):
            pltpu.sync_copy(rm_hbm.at[wid, g], idx)

            def step(t, _):
                pltpu.sync_copy(ones, d_sh.at[idx.at[t]], add=True)
                return 0
            lax.fori_loop(0, EG, step, 0)
            return 0
        lax.fori_loop(0, NGRP, group, 0)

        pltpu.sync_copy(rt_hbm.at[wid], tidx)
        pltpu.sync_copy(ones.at[pl.ds(0, TK)], d_sh.at[tidx], add=True)

        plsc.subcore_barrier()

        r0 = s * NPT
        pltpu.sync_copy(d_sh.at[pl.ds(r0, NPT)], d_out.at[c, pl.ds(r0, NPT)])

    fn = functools.partial(
        pl.kernel, mesh=mesh,
        out_type=jax.ShapeDtypeStruct((NC, N_PAD, L), _f32),
        scratch_types=[
            pltpu.VMEM((EG, EK), jnp.int32),
            pltpu.VMEM((TK,), jnp.int32),
            pltpu.VMEM((EK, L), _f32),
            pltpu.VMEM_SHARED((N_PAD, L), _f32),
        ],
    )(body)
    return fn(rm, rt)


def _sc_aggregate(feat, rm, cm, rt, ct):
    mesh = plsc.VectorSubcoreMesh(core_axis_name="c", subcore_axis_name="s")

    def body(feat_hbm, rm_hbm, cm_hbm, rt_hbm, ct_hbm, a_out,
             ridx, cidx, tridx, tcidx, rows, a_sh, sem):
        c = lax.axis_index("c")
        s = lax.axis_index("s")
        wid = c * NS + s

        _fill(rows, EK, F, 0.0)
        for z in range(NPT // EK):
            pltpu.sync_copy(rows, a_sh.at[pl.ds(s * NPT + z * EK, EK)])

        plsc.subcore_barrier()

        def group(g, _):
            pltpu.sync_copy(rm_hbm.at[wid, g], ridx)
            pltpu.sync_copy(cm_hbm.at[wid, g], cidx)

            def step(t, _):
                pltpu.async_copy(feat_hbm.at[cidx.at[t]], rows, sem).wait()
                pltpu.sync_copy(rows, a_sh.at[ridx.at[t]], add=True)
                return 0
            lax.fori_loop(0, EG, step, 0)
            return 0
        lax.fori_loop(0, NGRP, group, 0)

        pltpu.sync_copy(rt_hbm.at[wid], tridx)
        pltpu.sync_copy(ct_hbm.at[wid], tcidx)
        pltpu.async_copy(feat_hbm.at[tcidx], rows.at[pl.ds(0, TK)],
                         sem).wait()
        pltpu.sync_copy(rows.at[pl.ds(0, TK)], a_sh.at[tridx], add=True)

        plsc.subcore_barrier()

        r0 = s * NPT
        pltpu.sync_copy(a_sh.at[pl.ds(r0, NPT)], a_out.at[c, pl.ds(r0, NPT)])

    fn = functools.partial(
        pl.kernel, mesh=mesh,
        out_type=jax.ShapeDtypeStruct((NC, N_PAD, F), _f32),
        scratch_types=[
            pltpu.VMEM((EG, EK), jnp.int32),
            pltpu.VMEM((EG, EK), jnp.int32),
            pltpu.VMEM((TK,), jnp.int32),
            pltpu.VMEM((TK,), jnp.int32),
            pltpu.VMEM((EK, F), _f32),
            pltpu.VMEM_SHARED((N_PAD, F), _f32),
            pltpu.SemaphoreType.DMA,
        ],
    )(body)
    return fn(feat, rm, cm, rt, ct)


def _sc_pool(h, batch3d):
    mesh = plsc.VectorSubcoreMesh(core_axis_name="c", subcore_axis_name="s")

    def body(h_hbm, b_hbm, p_out, c_out, bidx, rows, ones, zp, zc, p_sh, c_sh,
             sem):
        c = lax.axis_index("c")
        s = lax.axis_index("s")
        wid = s * NC + c

        @pl.when(s == 0)
        def _zero():
            _fill(zp, G, F, 0.0)
            _fill(zc, G, L, 0.0)
            pltpu.sync_copy(zp, p_sh)
            pltpu.sync_copy(zc, c_sh)

        plsc.subcore_barrier()

        @pl.when(wid < PACT)
        def _accum():
            _fill(ones, PK, L, 1.0)
            pltpu.sync_copy(b_hbm.at[wid], bidx)

            def step(t, _):
                pltpu.async_copy(
                    h_hbm.at[pl.ds(wid * PNP + t * PK, PK)], rows, sem).wait()
                pltpu.sync_copy(rows, p_sh.at[bidx.at[t]], add=True)
                pltpu.sync_copy(ones, c_sh.at[bidx.at[t]], add=True)
                return 0
            lax.fori_loop(0, PSTEPS, step, 0)

        plsc.subcore_barrier()

        @pl.when(s == 0)
        def _write():
            pltpu.sync_copy(p_sh, p_out.at[c])
            pltpu.sync_copy(c_sh, c_out.at[c])

    fn = functools.partial(
        pl.kernel, mesh=mesh,
        out_type=(jax.ShapeDtypeStruct((NC, G, F), _f32),
                  jax.ShapeDtypeStruct((NC, G, L), _f32)),
        scratch_types=[
            pltpu.VMEM((PSTEPS, PK), jnp.int32),
            pltpu.VMEM((PK, F), _f32),
            pltpu.VMEM((PK, L), _f32),
            pltpu.VMEM((G, F), _f32),
            pltpu.VMEM((G, L), _f32),
            pltpu.VMEM_SHARED((G, F), _f32),
            pltpu.VMEM_SHARED((G, L), _f32),
            pltpu.SemaphoreType.DMA,
        ],
    )(body)
    return fn(h, batch3d)


BN = 512
NB = N_PAD // BN


def _tc_layer(a_part, deg_part, xin, Wm, bm, Wu_top, Wu_bot, bu):
    def body(a_ref, d_ref, x_ref, wm_ref, bm_ref, ut_ref, ub_ref, bu_ref,
             o_ref):
        a = a_ref[0] + a_ref[1]
        deg = d_ref[0, :, 0:1] + d_ref[1, :, 0:1]
        aggr = jnp.dot(a, wm_ref[:], preferred_element_type=_f32)
        aggr = aggr + deg * bm_ref[:]
        h = (jnp.dot(x_ref[:], ut_ref[:], preferred_element_type=_f32)
             + jnp.dot(aggr, ub_ref[:], preferred_element_type=_f32)
             + bu_ref[:])
        o_ref[:] = jnp.maximum(h, 0.0)

    return pl.pallas_call(
        body,
        grid=(NB,),
        in_specs=[
            pl.BlockSpec((NC, BN, F), lambda i: (0, i, 0)),
            pl.BlockSpec((NC, BN, L), lambda i: (0, i, 0)),
            pl.BlockSpec((BN, F), lambda i: (i, 0)),
            pl.BlockSpec((F, F), lambda i: (0, 0)),
            pl.BlockSpec((1, F), lambda i: (0, 0)),
            pl.BlockSpec((F, F), lambda i: (0, 0)),
            pl.BlockSpec((F, F), lambda i: (0, 0)),
            pl.BlockSpec((1, F), lambda i: (0, 0)),
        ],
        out_specs=pl.BlockSpec((BN, F), lambda i: (i, 0)),
        out_shape=jax.ShapeDtypeStruct((N_PAD, F), _f32),
    )(a_part, deg_part, xin, Wm, bm, Wu_top, Wu_bot, bu)


def _tc_head(p_part, c_part, Wc_pad, bc_pad):
    def body(p_ref, c_ref, wc_ref, bc_ref, o_ref):
        p = p_ref[0] + p_ref[1]
        cnt = c_ref[0, :, 0:1] + c_ref[1, :, 0:1]
        pooled = p / cnt
        o_ref[:] = (jnp.dot(pooled, wc_ref[:], preferred_element_type=_f32)
                    + bc_ref[:])

    return pl.pallas_call(
        body,
        out_shape=jax.ShapeDtypeStruct((G, F), _f32),
    )(p_part, c_part, Wc_pad, bc_pad)


def kernel(x, edge_index, batch, W1m, b1m, W1u, b1u, W2m, b2m, W2u, b2u,
           Wc, bc):
    row2 = edge_index[0].astype(jnp.int32).reshape(NW, EPT)
    col2 = edge_index[1].astype(jnp.int32).reshape(NW, EPT)
    nmain = NGRP * EG * EK
    rm = row2[:, :nmain].reshape(NW, NGRP, EG, EK)
    cm = col2[:, :nmain].reshape(NW, NGRP, EG, EK)
    rt = row2[:, nmain:]
    ct = col2[:, nmain:]
    batch3d = batch.astype(jnp.int32).reshape(PACT, PSTEPS, PK)

    b1m_2d = b1m.reshape(1, F)
    b1u_2d = b1u.reshape(1, F)
    b2m_2d = b2m.reshape(1, F)
    b2u_2d = b2u.reshape(1, F)
    Wc_pad = jnp.zeros((F, F), _f32).at[:, :NCLS].set(Wc)
    bc_pad = jnp.zeros((1, F), _f32).at[0, :NCLS].set(bc)

    xp = jnp.zeros((N_PAD, F), _f32).at[:N_NODES].set(x)

    deg = _sc_degree(rm, rt)
    a1 = _sc_aggregate(xp, rm, cm, rt, ct)
    h1 = _tc_layer(a1, deg, xp, W1m, b1m_2d, W1u[:F], W1u[F:], b1u_2d)
    a2 = _sc_aggregate(h1, rm, cm, rt, ct)
    h2 = _tc_layer(a2, deg, h1, W2m, b2m_2d, W2u[:F], W2u[F:], b2u_2d)
    p, cnt = _sc_pool(h2, batch3d)
    out = _tc_head(p, cnt, Wc_pad, bc_pad)
    return out[:, :NCLS]


# speedup vs baseline: 7.0401x; 1.0315x over previous
import functools

import jax
import jax.numpy as jnp
from jax import lax
from jax.experimental import pallas as pl
from jax.experimental.pallas import tpu as pltpu
from jax.experimental.pallas import tpu_sc as plsc

F = 128
N_NODES = 10000
N_PAD = 10240
N_EDGES = 320000
G = 64
NCLS = 10

NC = 2
NS = 16
NW = NC * NS
L = 16

EPT = N_EDGES // NW
EK = 128
EG = 13
NGRP = 6
TK = EPT - NGRP * EG * EK
NPT = N_PAD // NS

PACT = 25
PNP = N_NODES // PACT
PK = 80
PSTEPS = PNP // PK

_f32 = jnp.float32


def _fill(ref, rows, cols, value):
    def body(i, _):
        for j in range(cols // L):
            ref[i, pl.ds(j * L, L)] = jnp.full((L,), value, _f32)
        return 0
    lax.fori_loop(0, rows, body, 0)


def _sc_degree(rm, rt):
    mesh = plsc.VectorSubcoreMesh(core_axis_name="c", subcore_axis_name="s")

    def body(rm_hbm, rt_hbm, d_out, idx, tidx, ones, d_sh):
        c = lax.axis_index("c")
        s = lax.axis_index("s")
        wid = c * NS + s

        _fill(ones, EK, L, 0.0)
        for z in range(NPT // EK):
            pltpu.sync_copy(ones, d_sh.at[pl.ds(s * NPT + z * EK, EK)])
        _fill(ones, EK, L, 1.0)

        pltpu.sync_copy(rm_hbm.at[wid], idx)

        plsc.subcore_barrier()

        def group(g, _):
            def step(t, _):
                pltpu.sync_copy(ones, d_sh.at[idx.at[g, t]], add=True)
                return 0
            lax.fori_loop(0, EG, step, 0)
            return 0
        lax.fori_loop(0, NGRP, group, 0)

        pltpu.sync_copy(rt_hbm.at[wid], tidx)
        pltpu.sync_copy(ones.at[pl.ds(0, TK)], d_sh.at[tidx], add=True)

        plsc.subcore_barrier()

        r0 = s * NPT
        pltpu.sync_copy(d_sh.at[pl.ds(r0, NPT)], d_out.at[c, pl.ds(r0, NPT)])

    fn = functools.partial(
        pl.kernel, mesh=mesh,
        out_type=jax.ShapeDtypeStruct((NC, N_PAD, L), _f32),
        scratch_types=[
            pltpu.VMEM((NGRP, EG, EK), jnp.int32),
            pltpu.VMEM((TK,), jnp.int32),
            pltpu.VMEM((EK, L), _f32),
            pltpu.VMEM_SHARED((N_PAD, L), _f32),
        ],
    )(body)
    return fn(rm, rt)


def _sc_aggregate(feat, rm, cm, rt, ct):
    mesh = plsc.VectorSubcoreMesh(core_axis_name="c", subcore_axis_name="s")

    def body(feat_hbm, rm_hbm, cm_hbm, rt_hbm, ct_hbm, a_out,
             ridx, cidx, tridx, tcidx, rows, a_sh, sem):
        c = lax.axis_index("c")
        s = lax.axis_index("s")
        wid = c * NS + s

        _fill(rows, EK, F, 0.0)
        for z in range(NPT // EK):
            pltpu.sync_copy(rows, a_sh.at[pl.ds(s * NPT + z * EK, EK)])

        plsc.subcore_barrier()

        def group(g, _):
            pltpu.sync_copy(rm_hbm.at[wid, g], ridx)
            pltpu.sync_copy(cm_hbm.at[wid, g], cidx)

            def step(t, _):
                pltpu.async_copy(feat_hbm.at[cidx.at[t]], rows, sem).wait()
                pltpu.sync_copy(rows, a_sh.at[ridx.at[t]], add=True)
                return 0
            lax.fori_loop(0, EG, step, 0)
            return 0
        lax.fori_loop(0, NGRP, group, 0)

        pltpu.sync_copy(rt_hbm.at[wid], tridx)
        pltpu.sync_copy(ct_hbm.at[wid], tcidx)
        pltpu.async_copy(feat_hbm.at[tcidx], rows.at[pl.ds(0, TK)],
                         sem).wait()
        pltpu.sync_copy(rows.at[pl.ds(0, TK)], a_sh.at[tridx], add=True)

        plsc.subcore_barrier()

        r0 = s * NPT
        pltpu.sync_copy(a_sh.at[pl.ds(r0, NPT)], a_out.at[c, pl.ds(r0, NPT)])

    fn = functools.partial(
        pl.kernel, mesh=mesh,
        out_type=jax.ShapeDtypeStruct((NC, N_PAD, F), _f32),
        scratch_types=[
            pltpu.VMEM((EG, EK), jnp.int32),
            pltpu.VMEM((EG, EK), jnp.int32),
            pltpu.VMEM((TK,), jnp.int32),
            pltpu.VMEM((TK,), jnp.int32),
            pltpu.VMEM((EK, F), _f32),
            pltpu.VMEM_SHARED((N_PAD, F), _f32),
            pltpu.SemaphoreType.DMA,
        ],
    )(body)
    return fn(feat, rm, cm, rt, ct)


def _sc_pool(h, batch3d):
    mesh = plsc.VectorSubcoreMesh(core_axis_name="c", subcore_axis_name="s")

    def body(h_hbm, b_hbm, p_out, c_out, bidx, rows, ones, zp, zc, p_sh, c_sh,
             sem):
        c = lax.axis_index("c")
        s = lax.axis_index("s")
        wid = s * NC + c

        @pl.when(s == 0)
        def _zero():
            _fill(zp, G, F, 0.0)
            _fill(zc, G, L, 0.0)
            pltpu.sync_copy(zp, p_sh)
            pltpu.sync_copy(zc, c_sh)

        plsc.subcore_barrier()

        @pl.when(wid < PACT)
        def _accum():
            _fill(ones, PK, L, 1.0)
            pltpu.sync_copy(b_hbm.at[wid], bidx)

            def step(t, _):
                pltpu.async_copy(
                    h_hbm.at[pl.ds(wid * PNP + t * PK, PK)], rows, sem).wait()
                pltpu.sync_copy(rows, p_sh.at[bidx.at[t]], add=True)
                pltpu.sync_copy(ones, c_sh.at[bidx.at[t]], add=True)
                return 0
            lax.fori_loop(0, PSTEPS, step, 0)

        plsc.subcore_barrier()

        @pl.when(s == 0)
        def _write():
            pltpu.sync_copy(p_sh, p_out.at[c])
            pltpu.sync_copy(c_sh, c_out.at[c])

    fn = functools.partial(
        pl.kernel, mesh=mesh,
        out_type=(jax.ShapeDtypeStruct((NC, G, F), _f32),
                  jax.ShapeDtypeStruct((NC, G, L), _f32)),
        scratch_types=[
            pltpu.VMEM((PSTEPS, PK), jnp.int32),
            pltpu.VMEM((PK, F), _f32),
            pltpu.VMEM((PK, L), _f32),
            pltpu.VMEM((G, F), _f32),
            pltpu.VMEM((G, L), _f32),
            pltpu.VMEM_SHARED((G, F), _f32),
            pltpu.VMEM_SHARED((G, L), _f32),
            pltpu.SemaphoreType.DMA,
        ],
    )(body)
    return fn(h, batch3d)


BN = 1024
NB = N_PAD // BN


def _tc_layer(a_part, deg_part, xin, Wm, bm, Wu_top, Wu_bot, bu):
    def body(a_ref, d_ref, x_ref, wm_ref, bm_ref, ut_ref, ub_ref, bu_ref,
             o_ref):
        a = a_ref[0] + a_ref[1]
        deg = d_ref[0, :, 0:1] + d_ref[1, :, 0:1]
        aggr = jnp.dot(a, wm_ref[:], preferred_element_type=_f32)
        aggr = aggr + deg * bm_ref[:]
        h = (jnp.dot(x_ref[:], ut_ref[:], preferred_element_type=_f32)
             + jnp.dot(aggr, ub_ref[:], preferred_element_type=_f32)
             + bu_ref[:])
        o_ref[:] = jnp.maximum(h, 0.0)

    return pl.pallas_call(
        body,
        grid=(NB,),
        in_specs=[
            pl.BlockSpec((NC, BN, F), lambda i: (0, i, 0)),
            pl.BlockSpec((NC, BN, L), lambda i: (0, i, 0)),
            pl.BlockSpec((BN, F), lambda i: (i, 0)),
            pl.BlockSpec((F, F), lambda i: (0, 0)),
            pl.BlockSpec((1, F), lambda i: (0, 0)),
            pl.BlockSpec((F, F), lambda i: (0, 0)),
            pl.BlockSpec((F, F), lambda i: (0, 0)),
            pl.BlockSpec((1, F), lambda i: (0, 0)),
        ],
        out_specs=pl.BlockSpec((BN, F), lambda i: (i, 0)),
        out_shape=jax.ShapeDtypeStruct((N_PAD, F), _f32),
    )(a_part, deg_part, xin, Wm, bm, Wu_top, Wu_bot, bu)


def _tc_head(p_part, c_part, Wc_pad, bc_pad):
    def body(p_ref, c_ref, wc_ref, bc_ref, o_ref):
        p = p_ref[0] + p_ref[1]
        cnt = c_ref[0, :, 0:1] + c_ref[1, :, 0:1]
        pooled = p / cnt
        o_ref[:] = (jnp.dot(pooled, wc_ref[:], preferred_element_type=_f32)
                    + bc_ref[:])

    return pl.pallas_call(
        body,
        out_shape=jax.ShapeDtypeStruct((G, F), _f32),
    )(p_part, c_part, Wc_pad, bc_pad)


def kernel(x, edge_index, batch, W1m, b1m, W1u, b1u, W2m, b2m, W2u, b2u,
           Wc, bc):
    row2 = edge_index[0].astype(jnp.int32).reshape(NW, EPT)
    col2 = edge_index[1].astype(jnp.int32).reshape(NW, EPT)
    nmain = NGRP * EG * EK
    rm = row2[:, :nmain].reshape(NW, NGRP, EG, EK)
    cm = col2[:, :nmain].reshape(NW, NGRP, EG, EK)
    rt = row2[:, nmain:]
    ct = col2[:, nmain:]
    batch3d = batch.astype(jnp.int32).reshape(PACT, PSTEPS, PK)

    b1m_2d = b1m.reshape(1, F)
    b1u_2d = b1u.reshape(1, F)
    b2m_2d = b2m.reshape(1, F)
    b2u_2d = b2u.reshape(1, F)
    Wc_pad = jnp.zeros((F, F), _f32).at[:, :NCLS].set(Wc)
    bc_pad = jnp.zeros((1, F), _f32).at[0, :NCLS].set(bc)

    xp = jnp.zeros((N_PAD, F), _f32).at[:N_NODES].set(x)

    deg = _sc_degree(rm, rt)
    a1 = _sc_aggregate(xp, rm, cm, rt, ct)
    h1 = _tc_layer(a1, deg, xp, W1m, b1m_2d, W1u[:F], W1u[F:], b1u_2d)
    a2 = _sc_aggregate(h1, rm, cm, rt, ct)
    h2 = _tc_layer(a2, deg, h1, W2m, b2m_2d, W2u[:F], W2u[F:], b2u_2d)
    p, cnt = _sc_pool(h2, batch3d)
    out = _tc_head(p, cnt, Wc_pad, bc_pad)
    return out[:, :NCLS]
